# Initial kernel scaffold; baseline (speedup 1.0000x reference)
import jax
import jax.numpy as jnp
from jax.experimental import pallas as pl


def kernel(rank_id, suit_id, rank_emb, suit_emb):
    raise NotImplementedError("write your pallas kernel here")


# speedup vs baseline: 2.7996x; 2.7996x over previous
import functools

import jax
import jax.numpy as jnp
from jax import lax
from jax.experimental import pallas as pl
from jax.experimental.pallas import tpu as pltpu
from jax.experimental.pallas import tpu_sc as plsc

EMB_DIM = 128
BATCH = 16384
NUM_RANK = 15
NUM_SUIT = 5

NC = 2
NS = 16
L = 16
NW = NC * NS
BPW = BATCH // NW
CHUNK = 128
NCHUNK = BPW // CHUNK


def _combine_body(rank_ref, suit_ref, out_ref):
    out_ref[...] = rank_ref[...][:, None, :] + suit_ref[...][None, :, :]


_combine = pl.pallas_call(
    _combine_body,
    out_shape=jax.ShapeDtypeStruct((NUM_RANK, NUM_SUIT, EMB_DIM), jnp.float32),
)


@functools.partial(
    pl.kernel,
    mesh=plsc.VectorSubcoreMesh(core_axis_name="c", subcore_axis_name="s"),
    out_type=jax.ShapeDtypeStruct((BATCH, EMB_DIM), jnp.float32),
    scratch_types=[
        pltpu.VMEM((BPW,), jnp.int32),
        pltpu.VMEM((BPW,), jnp.int32),
        pltpu.VMEM((NCHUNK, CHUNK), jnp.int32),
        pltpu.VMEM((NCHUNK, CHUNK, EMB_DIM), jnp.float32),
        pltpu.SemaphoreType.DMA,
        pltpu.SemaphoreType.DMA,
    ],
)
def _sc_lookup(table_hbm, rank_hbm, suit_hbm, out_hbm,
               rank_v, suit_v, idx_v, rows_v, gsem, osem):
    wid = lax.axis_index("s") * NC + lax.axis_index("c")
    base = wid * BPW
    pltpu.sync_copy(rank_hbm.at[pl.ds(base, BPW)], rank_v)
    pltpu.sync_copy(suit_hbm.at[pl.ds(base, BPW)], suit_v)
    for i in range(BPW // L):
        j, c = divmod(i, CHUNK // L)
        r = rank_v[pl.ds(i * L, L)]
        s = suit_v[pl.ds(i * L, L)]
        idx_v[j, pl.ds(c * L, L)] = r * NUM_SUIT + s
    for j in range(NCHUNK):
        pltpu.async_copy(table_hbm.at[idx_v.at[j]], rows_v.at[j], gsem).wait()
        pltpu.sync_copy(rows_v.at[j], out_hbm.at[pl.ds(base + j * CHUNK, CHUNK)])


def kernel(rank_id, suit_id, rank_emb, suit_emb):
    comb = _combine(rank_emb, suit_emb).reshape(NUM_RANK * NUM_SUIT, EMB_DIM)
    return _sc_lookup(comb, rank_id.astype(jnp.int32), suit_id.astype(jnp.int32))
